# baseline (speedup 1.0000x reference)
import functools

import jax
import jax.numpy as jnp
from jax import lax
from jax.experimental import pallas as pl
from jax.experimental.pallas import tpu as pltpu
from jax.experimental.pallas import tpu_sc as plsc

K = 2
NW = 32
ROW_CH = 32


def _sigmoid(a):
    return 1.0 / (1.0 + jnp.exp(-a))


def _router_body(x_ref, gw_ref, w_ref, eid_ref, r_ref, counts_ref, starts_ref,
                 base_ref, *, E, BT):
    i = pl.program_id(0)

    @pl.when(i == 0)
    def _init():
        base_ref[...] = jnp.zeros_like(base_ref)

    x = x_ref[...]
    gw = gw_ref[...]
    logits = jnp.dot(x, gw, preferred_element_type=jnp.float32)
    m = jnp.max(logits, axis=-1, keepdims=True)
    ex = jnp.exp(logits - m)
    probs = ex / jnp.sum(ex, axis=-1, keepdims=True)
    iota = lax.broadcasted_iota(jnp.int32, probs.shape, 1)
    m1 = jnp.max(probs, axis=-1, keepdims=True)
    i1 = jnp.min(jnp.where(probs == m1, iota, E), axis=-1, keepdims=True)
    mask1 = iota == i1
    probs2 = jnp.where(mask1, -1.0, probs)
    m2 = jnp.max(probs2, axis=-1, keepdims=True)
    i2 = jnp.min(jnp.where(probs2 == m2, iota, E), axis=-1, keepdims=True)
    mask2 = iota == i2

    w_ref[...] = jnp.concatenate([m1, m2], axis=1)
    eid_ref[...] = jnp.concatenate([i1, i2], axis=1)

    oh1 = mask1.astype(jnp.float32)
    oh2 = mask2.astype(jnp.float32)
    oh = oh1 + oh2
    ri = lax.broadcasted_iota(jnp.int32, (BT, BT), 0)
    ci = lax.broadcasted_iota(jnp.int32, (BT, BT), 1)
    L = (ci < ri).astype(jnp.float32)
    cum = jnp.dot(L, oh, preferred_element_type=jnp.float32,
                  precision=lax.Precision.HIGHEST) + base_ref[...]
    r0 = jnp.sum(oh1 * cum, axis=-1, keepdims=True)
    r1 = jnp.sum(oh2 * cum, axis=-1, keepdims=True)
    r_ref[...] = jnp.concatenate([r0, r1], axis=1).astype(jnp.int32)

    newbase = base_ref[...] + jnp.sum(oh, axis=0, keepdims=True)
    base_ref[...] = newbase
    counts_ref[...] = newbase.astype(jnp.int32)
    fi = lax.broadcasted_iota(jnp.int32, (E, E), 0)
    ei = lax.broadcasted_iota(jnp.int32, (E, E), 1)
    U = (fi < ei).astype(jnp.float32)
    starts_ref[...] = jnp.dot(newbase, U, preferred_element_type=jnp.float32,
                              precision=lax.Precision.HIGHEST).astype(jnp.int32)


def _pos_body(eid_ref, r_ref, starts_ref, w_ref, pos_ref, wrep_ref, *, E):
    eid = eid_ref[...]
    r = r_ref[...]
    starts = starts_ref[...]
    iota = lax.broadcasted_iota(jnp.int32, (eid.shape[0], E), 1)
    p0 = jnp.sum(jnp.where(iota == eid[:, 0:1], starts, 0), axis=-1,
                 keepdims=True)
    p1 = jnp.sum(jnp.where(iota == eid[:, 1:2], starts, 0), axis=-1,
                 keepdims=True)
    pos_ref[...] = r + jnp.concatenate([p0, p1], axis=1)
    w = w_ref[...]
    BT2 = w.shape[0]
    wrep_ref[...] = jnp.broadcast_to(w[:, :, None],
                                     (BT2, K, 128)).reshape(BT2 * K, 128)


def _dispatch_body(pos_hbm, wrep_hbm, x_hbm, xa_hbm, ws_hbm,
                   pos_m, wrep_v, tok_m, rows_a, rows_b, semw, semg, sems,
                   *, PPW, D):
    wid = lax.axis_index("s") * 2 + lax.axis_index("c")
    base = wid * PPW
    nch = PPW // ROW_CH
    for ch in range(nch):
        pltpu.sync_copy(pos_hbm.at[pl.ds(base + ch * ROW_CH, ROW_CH)],
                        pos_m.at[ch])
    pltpu.sync_copy(wrep_hbm.at[pl.ds(base, PPW)], wrep_v)
    for j in range(PPW // 16):
        iv = lax.iota(jnp.int32, 16)
        tok_m[j // 2, pl.ds((j % 2) * 16, 16)] = (
            lax.shift_right_logical(base + j * 16 + iv, 1))
    cw = [pltpu.async_copy(wrep_v.at[pl.ds(ch * ROW_CH, ROW_CH)],
                           ws_hbm.at[pos_m.at[ch]], semw)
          for ch in range(nch)]
    bufs = (rows_a, rows_b)
    cg = [None] * nch
    cs = [None] * nch
    cg[0] = pltpu.async_copy(x_hbm.at[tok_m.at[0]], bufs[0], semg)
    for ch in range(nch):
        cg[ch].wait()
        cs[ch] = pltpu.async_copy(bufs[ch % 2], xa_hbm.at[pos_m.at[ch]], sems)
        if ch + 1 < nch:
            if ch >= 1:
                cs[ch - 1].wait()
            cg[ch + 1] = pltpu.async_copy(x_hbm.at[tok_m.at[ch + 1]],
                                          bufs[(ch + 1) % 2], semg)
    cs[nch - 1].wait()
    if nch >= 2:
        cs[nch - 2].wait()
    for c in cw:
        c.wait()


def _gmm_body(e_ref, mt_ref, lo_ref, hi_ref, first_ref,
              xa_ref, x_ref, wg_ref, wu_ref, wd_ref, bg_ref, bu_ref, bd_ref,
              ws_ref, ys_ref, *, BM, E):
    u = pl.program_id(0)
    e = e_ref[u]
    mt = mt_ref[u]
    lo = lo_ref[u]
    hi = hi_ref[u]
    first = first_ref[u]
    is_res = (e == E)
    xs = jnp.where(is_res, x_ref[...], xa_ref[...]).astype(jnp.bfloat16)
    row = mt * BM + lax.broadcasted_iota(jnp.int32, (BM, 1), 0)
    valid = (row >= lo) & (row < hi)
    a = jnp.dot(xs, wg_ref[0], preferred_element_type=jnp.float32) + bg_ref[0]
    uu = jnp.dot(xs, wu_ref[0], preferred_element_type=jnp.float32) + bu_ref[0]
    h = (a * _sigmoid(a)) * uu
    h = jnp.where(valid, h, 0.0).astype(jnp.bfloat16)
    y = jnp.dot(h, wd_ref[0], preferred_element_type=jnp.float32)
    y = y + jnp.where(valid, bd_ref[0], 0.0)
    wsv = ws_ref[...][:, 0:1]
    y = y * jnp.where(is_res, jnp.ones_like(wsv), wsv)

    @pl.when(first == 1)
    def _init():
        ys_ref[...] = y

    @pl.when(first == 0)
    def _acc():
        ys_ref[...] += y


def _combine_body(ys_hbm, pos_hbm, out_hbm,
                  pos_m, rows_a, rows_b, acc_a, acc_b, semg, semr, semo,
                  *, TPW, D, RES0):
    wid = lax.axis_index("s") * 2 + lax.axis_index("c")
    tbase = wid * TPW
    pbase = K * tbase
    nch = TPW // 16
    for ch in range(nch):
        pltpu.sync_copy(pos_hbm.at[pl.ds(pbase + ch * 32, 32)], pos_m.at[ch])
    rows = (rows_a, rows_b)
    accs = (acc_a, acc_b)

    def fire(ch):
        cg = pltpu.async_copy(ys_hbm.at[pos_m.at[ch]], rows[ch % 2], semg)
        cr = pltpu.async_copy(ys_hbm.at[pl.ds(RES0 + tbase + ch * 16, 16)],
                              accs[ch % 2], semr)
        return cg, cr

    pend = fire(0)
    co = [None] * nch
    for ch in range(nch):
        pend[0].wait()
        pend[1].wait()
        if ch + 1 < nch:
            if ch >= 1:
                co[ch - 1].wait()
            pend = fire(ch + 1)
        rv = rows[ch % 2]
        av = accs[ch % 2]

        def jbody(j, _, rv=rv, av=av):
            def dbody(dq, _):
                for q in range(4):
                    sl = pl.ds(dq * 64 + q * 16, 16)
                    av[j, sl] += rv[2 * j, sl] + rv[2 * j + 1, sl]
                return 0

            lax.fori_loop(0, D // 64, dbody, 0)
            return 0

        lax.fori_loop(0, 16, jbody, 0)
        co[ch] = pltpu.async_copy(av, out_hbm.at[pl.ds(tbase + ch * 16, 16)],
                                  semo)
    co[nch - 1].wait()
    if nch >= 2:
        co[nch - 2].wait()


def kernel(x, gate_W, W_gate, W_up, W_down, b_gate, b_up, b_down,
           Wr_gate, Wr_up, Wr_down, br_gate, br_up, br_down):
    T, D = x.shape
    E = gate_W.shape[1]
    HE = W_gate.shape[2]
    P = T * K

    BT = min(T, 512)
    w_pair, eid, rank, counts, starts_arr = pl.pallas_call(
        functools.partial(_router_body, E=E, BT=BT),
        grid=(T // BT,),
        in_specs=[
            pl.BlockSpec((BT, D), lambda i: (i, 0)),
            pl.BlockSpec((D, E), lambda i: (0, 0)),
        ],
        out_specs=[
            pl.BlockSpec((BT, K), lambda i: (i, 0)),
            pl.BlockSpec((BT, K), lambda i: (i, 0)),
            pl.BlockSpec((BT, K), lambda i: (i, 0)),
            pl.BlockSpec((1, E), lambda i: (0, 0)),
            pl.BlockSpec((1, E), lambda i: (0, 0)),
        ],
        out_shape=[
            jax.ShapeDtypeStruct((T, K), jnp.float32),
            jax.ShapeDtypeStruct((T, K), jnp.int32),
            jax.ShapeDtypeStruct((T, K), jnp.int32),
            jax.ShapeDtypeStruct((1, E), jnp.int32),
            jax.ShapeDtypeStruct((1, E), jnp.int32),
        ],
        scratch_shapes=[pltpu.VMEM((1, E), jnp.float32)],
    )(x, gate_W)

    BT2 = min(T, 2048)
    pos, wrep = pl.pallas_call(
        functools.partial(_pos_body, E=E),
        grid=(T // BT2,),
        in_specs=[
            pl.BlockSpec((BT2, K), lambda i: (i, 0)),
            pl.BlockSpec((BT2, K), lambda i: (i, 0)),
            pl.BlockSpec((1, E), lambda i: (0, 0)),
            pl.BlockSpec((BT2, K), lambda i: (i, 0)),
        ],
        out_specs=[
            pl.BlockSpec((BT2, K), lambda i: (i, 0)),
            pl.BlockSpec((BT2 * K, 128), lambda i: (i, 0)),
        ],
        out_shape=[
            jax.ShapeDtypeStruct((T, K), jnp.int32),
            jax.ShapeDtypeStruct((P, 128), jnp.float32),
        ],
    )(eid, rank, starts_arr, w_pair)

    pos = pos.reshape(P)
    xa, ws = _run_dispatch(pos, wrep, x)

    BM = 1024
    MT_S = P // BM
    MT_R = T // BM
    G_MOE = MT_S + E - 1
    cnt = counts.reshape(E)
    starts = jnp.cumsum(cnt) - cnt
    ends = starts + cnt
    first_tile = starts // BM
    last_tile = jnp.maximum(ends - 1, 0) // BM
    mt_ar = jnp.arange(MT_S)[:, None]
    ov = ((mt_ar >= first_tile[None, :]) & (mt_ar <= last_tile[None, :])
          & (cnt > 0)[None, :])
    flat = ov.reshape(-1)
    idx = jnp.nonzero(flat, size=G_MOE, fill_value=0)[0]
    nreal = jnp.sum(flat.astype(jnp.int32))
    uvalid = jnp.arange(G_MOE) < nreal
    mtu = idx // E
    eu = idx % E
    lo = jnp.maximum(starts[eu], mtu * BM)
    hi = jnp.minimum(ends[eu], (mtu + 1) * BM)
    mtu = jnp.where(uvalid, mtu, MT_S - 1)
    eu = jnp.where(uvalid, eu, 0)
    lo = jnp.where(uvalid, lo, 0)
    hi = jnp.where(uvalid, hi, 0)
    mtr = MT_S + jnp.arange(MT_R)
    e_arr = jnp.concatenate([eu, jnp.full((MT_R,), E)]).astype(jnp.int32)
    mt_arr = jnp.concatenate([mtu, mtr]).astype(jnp.int32)
    lo_arr = jnp.concatenate([lo, mtr * BM]).astype(jnp.int32)
    hi_arr = jnp.concatenate([hi, (mtr + 1) * BM]).astype(jnp.int32)
    first_arr = jnp.concatenate(
        [jnp.array([1]), (mt_arr[1:] != mt_arr[:-1]).astype(jnp.int32)])
    G = G_MOE + MT_R

    Wg_all = jnp.concatenate([W_gate, Wr_gate[None]], axis=0).astype(jnp.bfloat16)
    Wu_all = jnp.concatenate([W_up, Wr_up[None]], axis=0).astype(jnp.bfloat16)
    Wd_all = jnp.concatenate([W_down, Wr_down[None]], axis=0).astype(jnp.bfloat16)
    bg_all = jnp.concatenate([b_gate, br_gate[None]], axis=0).reshape(E + 1, 1, HE)
    bu_all = jnp.concatenate([b_up, br_up[None]], axis=0).reshape(E + 1, 1, HE)
    bd_all = jnp.concatenate([b_down, br_down[None]], axis=0).reshape(E + 1, 1, D)

    ys = pl.pallas_call(
        functools.partial(_gmm_body, BM=BM, E=E),
        grid_spec=pltpu.PrefetchScalarGridSpec(
            num_scalar_prefetch=5,
            grid=(G,),
            in_specs=[
                pl.BlockSpec((BM, D),
                             lambda u, es, mts, los, his, fs:
                             (jnp.minimum(mts[u], MT_S - 1), 0)),
                pl.BlockSpec((BM, D),
                             lambda u, es, mts, los, his, fs:
                             (jnp.maximum(mts[u] - MT_S, 0), 0)),
                pl.BlockSpec((1, D, HE),
                             lambda u, es, mts, los, his, fs: (es[u], 0, 0)),
                pl.BlockSpec((1, D, HE),
                             lambda u, es, mts, los, his, fs: (es[u], 0, 0)),
                pl.BlockSpec((1, HE, D),
                             lambda u, es, mts, los, his, fs: (es[u], 0, 0)),
                pl.BlockSpec((1, 1, HE),
                             lambda u, es, mts, los, his, fs: (es[u], 0, 0)),
                pl.BlockSpec((1, 1, HE),
                             lambda u, es, mts, los, his, fs: (es[u], 0, 0)),
                pl.BlockSpec((1, 1, D),
                             lambda u, es, mts, los, his, fs: (es[u], 0, 0)),
                pl.BlockSpec((BM, 128),
                             lambda u, es, mts, los, his, fs:
                             (jnp.minimum(mts[u], MT_S - 1), 0)),
            ],
            out_specs=pl.BlockSpec((BM, D),
                                   lambda u, es, mts, los, his, fs:
                                   (mts[u], 0)),
        ),
        out_shape=jax.ShapeDtypeStruct((P + T, D), jnp.float32),
        compiler_params=pltpu.CompilerParams(
            dimension_semantics=("arbitrary",)),
    )(e_arr, mt_arr, lo_arr, hi_arr, first_arr,
      xa, x, Wg_all, Wu_all, Wd_all, bg_all, bu_all, bd_all, ws)

    out = _run_combine(ys, pos)
    return out


def _run_dispatch(posf, wrep, x):
    P = posf.shape[0]
    T, D = x.shape
    PPW = P // NW
    dispatch = pl.kernel(
        functools.partial(_dispatch_body, PPW=PPW, D=D),
        out_type=[
            jax.ShapeDtypeStruct((P, D), jnp.float32),
            jax.ShapeDtypeStruct((P, 128), jnp.float32),
        ],
        mesh=plsc.VectorSubcoreMesh(core_axis_name="c", subcore_axis_name="s"),
        scratch_types=[
            pltpu.VMEM((PPW // ROW_CH, ROW_CH), jnp.int32),
            pltpu.VMEM((PPW, 128), jnp.float32),
            pltpu.VMEM((PPW // ROW_CH, ROW_CH), jnp.int32),
            pltpu.VMEM((ROW_CH, D), jnp.float32),
            pltpu.VMEM((ROW_CH, D), jnp.float32),
            pltpu.SemaphoreType.DMA,
            pltpu.SemaphoreType.DMA,
            pltpu.SemaphoreType.DMA,
        ],
    )
    return dispatch(posf, wrep, x)


def _run_combine(ys, pos):
    P = pos.shape[0]
    T = P // K
    D = ys.shape[1]
    TPW = T // NW
    combine = pl.kernel(
        functools.partial(_combine_body, TPW=TPW, D=D, RES0=P),
        out_type=jax.ShapeDtypeStruct((T, D), jnp.float32),
        mesh=plsc.VectorSubcoreMesh(core_axis_name="c", subcore_axis_name="s"),
        scratch_types=[
            pltpu.VMEM((TPW // 16, 32), jnp.int32),
            pltpu.VMEM((32, D), jnp.float32),
            pltpu.VMEM((32, D), jnp.float32),
            pltpu.VMEM((16, D), jnp.float32),
            pltpu.VMEM((16, D), jnp.float32),
            pltpu.SemaphoreType.DMA,
            pltpu.SemaphoreType.DMA,
            pltpu.SemaphoreType.DMA,
        ],
    )
    return combine(ys, pos)


# speedup vs baseline: 1.0521x; 1.0521x over previous
import functools

import jax
import jax.numpy as jnp
from jax import lax
from jax.experimental import pallas as pl
from jax.experimental.pallas import tpu as pltpu
from jax.experimental.pallas import tpu_sc as plsc

K = 2
NW = 32
ROW_CH = 32


def _sigmoid(a):
    return 1.0 / (1.0 + jnp.exp(-a))


def _router_body(x_ref, gw_ref, w_ref, eid_ref, r_ref, counts_ref, starts_ref,
                 base_ref, *, E, BT):
    i = pl.program_id(0)

    @pl.when(i == 0)
    def _init():
        base_ref[...] = jnp.zeros_like(base_ref)

    x = x_ref[...]
    gw = gw_ref[...]
    logits = jnp.dot(x, gw, preferred_element_type=jnp.float32)
    m = jnp.max(logits, axis=-1, keepdims=True)
    ex = jnp.exp(logits - m)
    probs = ex / jnp.sum(ex, axis=-1, keepdims=True)
    iota = lax.broadcasted_iota(jnp.int32, probs.shape, 1)
    m1 = jnp.max(probs, axis=-1, keepdims=True)
    i1 = jnp.min(jnp.where(probs == m1, iota, E), axis=-1, keepdims=True)
    mask1 = iota == i1
    probs2 = jnp.where(mask1, -1.0, probs)
    m2 = jnp.max(probs2, axis=-1, keepdims=True)
    i2 = jnp.min(jnp.where(probs2 == m2, iota, E), axis=-1, keepdims=True)
    mask2 = iota == i2

    w_ref[...] = jnp.concatenate([m1, m2], axis=1)
    eid_ref[...] = jnp.concatenate([i1, i2], axis=1)

    oh1 = mask1.astype(jnp.float32)
    oh2 = mask2.astype(jnp.float32)
    oh = oh1 + oh2
    ri = lax.broadcasted_iota(jnp.int32, (BT, BT), 0)
    ci = lax.broadcasted_iota(jnp.int32, (BT, BT), 1)
    L = (ci < ri).astype(jnp.float32)
    cum = jnp.dot(L, oh, preferred_element_type=jnp.float32) + base_ref[...]
    r0 = jnp.sum(oh1 * cum, axis=-1, keepdims=True)
    r1 = jnp.sum(oh2 * cum, axis=-1, keepdims=True)
    r_ref[...] = jnp.concatenate([r0, r1], axis=1).astype(jnp.int32)

    newbase = base_ref[...] + jnp.sum(oh, axis=0, keepdims=True)
    base_ref[...] = newbase
    counts_ref[...] = newbase.astype(jnp.int32)
    fi = lax.broadcasted_iota(jnp.int32, (E, E), 0)
    ei = lax.broadcasted_iota(jnp.int32, (E, E), 1)
    U = (fi < ei).astype(jnp.float32)
    starts_ref[...] = jnp.dot(newbase, U, preferred_element_type=jnp.float32,
                              precision=lax.Precision.HIGHEST).astype(jnp.int32)


def _pos_body(eid_ref, r_ref, starts_ref, w_ref, pos_ref, wrep_ref, *, E):
    eid = eid_ref[...]
    r = r_ref[...]
    starts = starts_ref[...]
    iota = lax.broadcasted_iota(jnp.int32, (eid.shape[0], E), 1)
    p0 = jnp.sum(jnp.where(iota == eid[:, 0:1], starts, 0), axis=-1,
                 keepdims=True)
    p1 = jnp.sum(jnp.where(iota == eid[:, 1:2], starts, 0), axis=-1,
                 keepdims=True)
    pos_ref[...] = r + jnp.concatenate([p0, p1], axis=1)
    w = w_ref[...]
    BT2 = w.shape[0]
    wrep_ref[...] = jnp.broadcast_to(w[:, :, None],
                                     (BT2, K, 128)).reshape(BT2 * K, 128)


def _dispatch_body(pos_hbm, wrep_hbm, x_hbm, xa_hbm, ws_hbm,
                   pos_m, wrep_v, tok_m, rows_a, rows_b, semw, semg, sems,
                   *, PPW, D):
    wid = lax.axis_index("s") * 2 + lax.axis_index("c")
    base = wid * PPW
    nch = PPW // ROW_CH
    for ch in range(nch):
        pltpu.sync_copy(pos_hbm.at[pl.ds(base + ch * ROW_CH, ROW_CH)],
                        pos_m.at[ch])
    pltpu.sync_copy(wrep_hbm.at[pl.ds(base, PPW)], wrep_v)
    for j in range(PPW // 16):
        iv = lax.iota(jnp.int32, 16)
        tok_m[j // 2, pl.ds((j % 2) * 16, 16)] = (
            lax.shift_right_logical(base + j * 16 + iv, 1))
    cw = [pltpu.async_copy(wrep_v.at[pl.ds(ch * ROW_CH, ROW_CH)],
                           ws_hbm.at[pos_m.at[ch]], semw)
          for ch in range(nch)]
    bufs = (rows_a, rows_b)
    cg = [None] * nch
    cs = [None] * nch
    cg[0] = pltpu.async_copy(x_hbm.at[tok_m.at[0]], bufs[0], semg)
    for ch in range(nch):
        cg[ch].wait()
        cs[ch] = pltpu.async_copy(bufs[ch % 2], xa_hbm.at[pos_m.at[ch]], sems)
        if ch + 1 < nch:
            if ch >= 1:
                cs[ch - 1].wait()
            cg[ch + 1] = pltpu.async_copy(x_hbm.at[tok_m.at[ch + 1]],
                                          bufs[(ch + 1) % 2], semg)
    cs[nch - 1].wait()
    if nch >= 2:
        cs[nch - 2].wait()
    for c in cw:
        c.wait()


def _gmm_body(e_ref, mt_ref, lo_ref, hi_ref, first_ref,
              xa_ref, x_ref, wg_ref, wu_ref, wd_ref, bg_ref, bu_ref, bd_ref,
              ws_ref, ys_ref, *, BM, E):
    u = pl.program_id(0)
    e = e_ref[u]
    mt = mt_ref[u]
    lo = lo_ref[u]
    hi = hi_ref[u]
    first = first_ref[u]
    is_res = (e == E)
    xs = jnp.where(is_res, x_ref[...], xa_ref[...]).astype(jnp.bfloat16)
    row = mt * BM + lax.broadcasted_iota(jnp.int32, (BM, 1), 0)
    valid = (row >= lo) & (row < hi)
    a = jnp.dot(xs, wg_ref[0], preferred_element_type=jnp.float32) + bg_ref[0]
    uu = jnp.dot(xs, wu_ref[0], preferred_element_type=jnp.float32) + bu_ref[0]
    h = (a * _sigmoid(a)) * uu
    h = jnp.where(valid, h, 0.0).astype(jnp.bfloat16)
    y = jnp.dot(h, wd_ref[0], preferred_element_type=jnp.float32)
    y = y + jnp.where(valid, bd_ref[0], 0.0)
    wsv = ws_ref[...][:, 0:1]
    y = y * jnp.where(is_res, jnp.ones_like(wsv), wsv)

    @pl.when(first == 1)
    def _init():
        ys_ref[...] = y

    @pl.when(first == 0)
    def _acc():
        ys_ref[...] += y


def _combine_body(ys_hbm, pos_hbm, out_hbm,
                  pos_m, rows_a, rows_b, acc_a, acc_b, semg, semr, semo,
                  *, TPW, D, RES0):
    wid = lax.axis_index("s") * 2 + lax.axis_index("c")
    tbase = wid * TPW
    pbase = K * tbase
    nch = TPW // 16
    for ch in range(nch):
        pltpu.sync_copy(pos_hbm.at[pl.ds(pbase + ch * 32, 32)], pos_m.at[ch])
    rows = (rows_a, rows_b)
    accs = (acc_a, acc_b)

    def fire(ch):
        cg = pltpu.async_copy(ys_hbm.at[pos_m.at[ch]], rows[ch % 2], semg)
        cr = pltpu.async_copy(ys_hbm.at[pl.ds(RES0 + tbase + ch * 16, 16)],
                              accs[ch % 2], semr)
        return cg, cr

    pend = fire(0)
    co = [None] * nch
    for ch in range(nch):
        pend[0].wait()
        pend[1].wait()
        if ch + 1 < nch:
            if ch >= 1:
                co[ch - 1].wait()
            pend = fire(ch + 1)
        rv = rows[ch % 2]
        av = accs[ch % 2]

        def jbody(j, _, rv=rv, av=av):
            def dbody(dq, _):
                for q in range(4):
                    sl = pl.ds(dq * 64 + q * 16, 16)
                    av[j, sl] += rv[2 * j, sl] + rv[2 * j + 1, sl]
                return 0

            lax.fori_loop(0, D // 64, dbody, 0)
            return 0

        lax.fori_loop(0, 16, jbody, 0)
        co[ch] = pltpu.async_copy(av, out_hbm.at[pl.ds(tbase + ch * 16, 16)],
                                  semo)
    co[nch - 1].wait()
    if nch >= 2:
        co[nch - 2].wait()


def kernel(x, gate_W, W_gate, W_up, W_down, b_gate, b_up, b_down,
           Wr_gate, Wr_up, Wr_down, br_gate, br_up, br_down):
    T, D = x.shape
    E = gate_W.shape[1]
    HE = W_gate.shape[2]
    P = T * K

    BT = min(T, 512)
    w_pair, eid, rank, counts, starts_arr = pl.pallas_call(
        functools.partial(_router_body, E=E, BT=BT),
        grid=(T // BT,),
        in_specs=[
            pl.BlockSpec((BT, D), lambda i: (i, 0)),
            pl.BlockSpec((D, E), lambda i: (0, 0)),
        ],
        out_specs=[
            pl.BlockSpec((BT, K), lambda i: (i, 0)),
            pl.BlockSpec((BT, K), lambda i: (i, 0)),
            pl.BlockSpec((BT, K), lambda i: (i, 0)),
            pl.BlockSpec((1, E), lambda i: (0, 0)),
            pl.BlockSpec((1, E), lambda i: (0, 0)),
        ],
        out_shape=[
            jax.ShapeDtypeStruct((T, K), jnp.float32),
            jax.ShapeDtypeStruct((T, K), jnp.int32),
            jax.ShapeDtypeStruct((T, K), jnp.int32),
            jax.ShapeDtypeStruct((1, E), jnp.int32),
            jax.ShapeDtypeStruct((1, E), jnp.int32),
        ],
        scratch_shapes=[pltpu.VMEM((1, E), jnp.float32)],
    )(x, gate_W)

    BT2 = min(T, 2048)
    pos, wrep = pl.pallas_call(
        functools.partial(_pos_body, E=E),
        grid=(T // BT2,),
        in_specs=[
            pl.BlockSpec((BT2, K), lambda i: (i, 0)),
            pl.BlockSpec((BT2, K), lambda i: (i, 0)),
            pl.BlockSpec((1, E), lambda i: (0, 0)),
            pl.BlockSpec((BT2, K), lambda i: (i, 0)),
        ],
        out_specs=[
            pl.BlockSpec((BT2, K), lambda i: (i, 0)),
            pl.BlockSpec((BT2 * K, 128), lambda i: (i, 0)),
        ],
        out_shape=[
            jax.ShapeDtypeStruct((T, K), jnp.int32),
            jax.ShapeDtypeStruct((P, 128), jnp.float32),
        ],
    )(eid, rank, starts_arr, w_pair)

    pos = pos.reshape(P)
    xa, ws = _run_dispatch(pos, wrep, x)

    BM = 512
    MT_S = P // BM
    MT_R = T // BM
    G_MOE = MT_S + E - 1
    cnt = counts.reshape(E)
    starts = jnp.cumsum(cnt) - cnt
    ends = starts + cnt
    first_tile = starts // BM
    last_tile = jnp.maximum(ends - 1, 0) // BM
    mt_ar = jnp.arange(MT_S)[:, None]
    ov = ((mt_ar >= first_tile[None, :]) & (mt_ar <= last_tile[None, :])
          & (cnt > 0)[None, :])
    flat = ov.reshape(-1)
    idx = jnp.nonzero(flat, size=G_MOE, fill_value=0)[0]
    nreal = jnp.sum(flat.astype(jnp.int32))
    uvalid = jnp.arange(G_MOE) < nreal
    mtu = idx // E
    eu = idx % E
    lo = jnp.maximum(starts[eu], mtu * BM)
    hi = jnp.minimum(ends[eu], (mtu + 1) * BM)
    mtu = jnp.where(uvalid, mtu, MT_S - 1)
    eu = jnp.where(uvalid, eu, 0)
    lo = jnp.where(uvalid, lo, 0)
    hi = jnp.where(uvalid, hi, 0)
    mtr = MT_S + jnp.arange(MT_R)
    e_arr = jnp.concatenate([eu, jnp.full((MT_R,), E)]).astype(jnp.int32)
    mt_arr = jnp.concatenate([mtu, mtr]).astype(jnp.int32)
    lo_arr = jnp.concatenate([lo, mtr * BM]).astype(jnp.int32)
    hi_arr = jnp.concatenate([hi, (mtr + 1) * BM]).astype(jnp.int32)
    first_arr = jnp.concatenate(
        [jnp.array([1]), (mt_arr[1:] != mt_arr[:-1]).astype(jnp.int32)])
    G = G_MOE + MT_R

    Wg_all = jnp.concatenate([W_gate, Wr_gate[None]], axis=0).astype(jnp.bfloat16)
    Wu_all = jnp.concatenate([W_up, Wr_up[None]], axis=0).astype(jnp.bfloat16)
    Wd_all = jnp.concatenate([W_down, Wr_down[None]], axis=0).astype(jnp.bfloat16)
    bg_all = jnp.concatenate([b_gate, br_gate[None]], axis=0).reshape(E + 1, 1, HE)
    bu_all = jnp.concatenate([b_up, br_up[None]], axis=0).reshape(E + 1, 1, HE)
    bd_all = jnp.concatenate([b_down, br_down[None]], axis=0).reshape(E + 1, 1, D)

    ys = pl.pallas_call(
        functools.partial(_gmm_body, BM=BM, E=E),
        grid_spec=pltpu.PrefetchScalarGridSpec(
            num_scalar_prefetch=5,
            grid=(G,),
            in_specs=[
                pl.BlockSpec((BM, D),
                             lambda u, es, mts, los, his, fs:
                             (jnp.minimum(mts[u], MT_S - 1), 0)),
                pl.BlockSpec((BM, D),
                             lambda u, es, mts, los, his, fs:
                             (jnp.maximum(mts[u] - MT_S, 0), 0)),
                pl.BlockSpec((1, D, HE),
                             lambda u, es, mts, los, his, fs: (es[u], 0, 0)),
                pl.BlockSpec((1, D, HE),
                             lambda u, es, mts, los, his, fs: (es[u], 0, 0)),
                pl.BlockSpec((1, HE, D),
                             lambda u, es, mts, los, his, fs: (es[u], 0, 0)),
                pl.BlockSpec((1, 1, HE),
                             lambda u, es, mts, los, his, fs: (es[u], 0, 0)),
                pl.BlockSpec((1, 1, HE),
                             lambda u, es, mts, los, his, fs: (es[u], 0, 0)),
                pl.BlockSpec((1, 1, D),
                             lambda u, es, mts, los, his, fs: (es[u], 0, 0)),
                pl.BlockSpec((BM, 128),
                             lambda u, es, mts, los, his, fs:
                             (jnp.minimum(mts[u], MT_S - 1), 0)),
            ],
            out_specs=pl.BlockSpec((BM, D),
                                   lambda u, es, mts, los, his, fs:
                                   (mts[u], 0)),
        ),
        out_shape=jax.ShapeDtypeStruct((P + T, D), jnp.float32),
        compiler_params=pltpu.CompilerParams(
            dimension_semantics=("arbitrary",)),
    )(e_arr, mt_arr, lo_arr, hi_arr, first_arr,
      xa, x, Wg_all, Wu_all, Wd_all, bg_all, bu_all, bd_all, ws)

    out = _run_combine(ys, pos)
    return out


def _run_dispatch(posf, wrep, x):
    P = posf.shape[0]
    T, D = x.shape
    PPW = P // NW
    dispatch = pl.kernel(
        functools.partial(_dispatch_body, PPW=PPW, D=D),
        out_type=[
            jax.ShapeDtypeStruct((P, D), jnp.float32),
            jax.ShapeDtypeStruct((P, 128), jnp.float32),
        ],
        mesh=plsc.VectorSubcoreMesh(core_axis_name="c", subcore_axis_name="s"),
        scratch_types=[
            pltpu.VMEM((PPW // ROW_CH, ROW_CH), jnp.int32),
            pltpu.VMEM((PPW, 128), jnp.float32),
            pltpu.VMEM((PPW // ROW_CH, ROW_CH), jnp.int32),
            pltpu.VMEM((ROW_CH, D), jnp.float32),
            pltpu.VMEM((ROW_CH, D), jnp.float32),
            pltpu.SemaphoreType.DMA,
            pltpu.SemaphoreType.DMA,
            pltpu.SemaphoreType.DMA,
        ],
    )
    return dispatch(posf, wrep, x)


def _run_combine(ys, pos):
    P = pos.shape[0]
    T = P // K
    D = ys.shape[1]
    TPW = T // NW
    combine = pl.kernel(
        functools.partial(_combine_body, TPW=TPW, D=D, RES0=P),
        out_type=jax.ShapeDtypeStruct((T, D), jnp.float32),
        mesh=plsc.VectorSubcoreMesh(core_axis_name="c", subcore_axis_name="s"),
        scratch_types=[
            pltpu.VMEM((TPW // 16, 32), jnp.int32),
            pltpu.VMEM((32, D), jnp.float32),
            pltpu.VMEM((32, D), jnp.float32),
            pltpu.VMEM((16, D), jnp.float32),
            pltpu.VMEM((16, D), jnp.float32),
            pltpu.SemaphoreType.DMA,
            pltpu.SemaphoreType.DMA,
            pltpu.SemaphoreType.DMA,
        ],
    )
    return combine(ys, pos)
